# Initial kernel scaffold; baseline (speedup 1.0000x reference)
import jax
import jax.numpy as jnp
from jax.experimental import pallas as pl


def kernel(x, table):
    raise NotImplementedError("write your pallas kernel here")


# speedup vs baseline: 1.5248x; 1.5248x over previous
import functools

import jax
import jax.numpy as jnp
from jax import lax
from jax.experimental import pallas as pl
from jax.experimental.pallas import tpu as pltpu
from jax.experimental.pallas import tpu_sc as plsc

D_MODEL = 1024
SCALE = 32.0
LANES = 16

NC = 2
NS = 16
NW = NC * NS

B_TOTAL = 4 * 4096
B_PER_W = B_TOTAL // NW
CHUNK = 16
NCH = B_PER_W // CHUNK

_mesh = plsc.VectorSubcoreMesh(core_axis_name="c", subcore_axis_name="s")


def _scale_chunk(src, dst):
  def row(r, _):
    def col(i, _):
      sl = pl.ds(i * LANES, LANES)
      dst[r, sl] = src[r, sl] * SCALE
      return 0
    return lax.fori_loop(0, D_MODEL // LANES, col, 0, unroll=4)
  lax.fori_loop(0, CHUNK, row, 0)


@functools.partial(
    pl.kernel,
    out_type=jax.ShapeDtypeStruct((B_TOTAL, D_MODEL), jnp.float32),
    mesh=_mesh,
    scratch_types=[
        pltpu.VMEM((NCH, CHUNK), jnp.int32),
        pltpu.VMEM((CHUNK, D_MODEL), jnp.float32),
        pltpu.VMEM((CHUNK, D_MODEL), jnp.float32),
        pltpu.VMEM((CHUNK, D_MODEL), jnp.float32),
        pltpu.VMEM((CHUNK, D_MODEL), jnp.float32),
        pltpu.SemaphoreType.DMA,
        pltpu.SemaphoreType.DMA,
        pltpu.SemaphoreType.DMA,
        pltpu.SemaphoreType.DMA,
    ],
)
def _emb_lookup(x_hbm, table_hbm, out_hbm, idx_v, in0, in1, o0, o1,
                g0, g1, s0, s1):
  ins = (in0, in1)
  outs = (o0, o1)
  gsems = (g0, g1)
  ssems = (s0, s1)

  wid = lax.axis_index("s") * NC + lax.axis_index("c")
  base = wid * B_PER_W

  pltpu.sync_copy(x_hbm.at[wid], idx_v)

  for b in range(2):
    pltpu.async_copy(table_hbm.at[idx_v.at[b]], ins[b], gsems[b])

  for b in range(2):
    pltpu.make_async_copy(table_hbm.at[idx_v.at[b]], ins[b], gsems[b]).wait()
    _scale_chunk(ins[b], outs[b])
    pltpu.async_copy(
        outs[b], out_hbm.at[pl.ds(base + b * CHUNK, CHUNK)], ssems[b])
    pltpu.async_copy(table_hbm.at[idx_v.at[2 + b]], ins[b], gsems[b])

  def group(g, _):
    for b in range(2):
      j = g * 2 + b
      pltpu.make_async_copy(table_hbm.at[idx_v.at[j]], ins[b], gsems[b]).wait()
      pltpu.make_async_copy(
          outs[b], out_hbm.at[pl.ds(base, CHUNK)], ssems[b]).wait()
      _scale_chunk(ins[b], outs[b])
      pltpu.async_copy(
          outs[b], out_hbm.at[pl.ds(base + j * CHUNK, CHUNK)], ssems[b])
      nj = j + 2

      @pl.when(nj < NCH)
      def _():
        pltpu.async_copy(table_hbm.at[idx_v.at[nj]], ins[b], gsems[b])
    return 0

  lax.fori_loop(1, NCH // 2, group, 0)

  for b in range(2):
    pltpu.make_async_copy(
        outs[b], out_hbm.at[pl.ds(base, CHUNK)], ssems[b]).wait()


def kernel(x, table):
  xf = x.astype(jnp.int32).reshape(NW, NCH, CHUNK)
  out = _emb_lookup(xf, table)
  return out.reshape(x.shape + (D_MODEL,))
